# Initial kernel scaffold; baseline (speedup 1.0000x reference)
import jax
import jax.numpy as jnp
from jax.experimental import pallas as pl


def kernel(x, W_enc, b_enc, W_dec, b_dec):
    raise NotImplementedError("write your pallas kernel here")


# speedup vs baseline: 5.2144x; 5.2144x over previous
import jax
import jax.numpy as jnp
from jax import lax
from jax.experimental import pallas as pl
from jax.experimental.pallas import tpu as pltpu

ROWS = 8192
D_IN = 768
D_LAT = 12288
K = 20
BLK = 128
CHUNK = 1024
NC = D_LAT // CHUNK
NEG = -3.4e38


def _enc_body(x_ref, we_ref, be_ref, sp_ref, lat_ref):
    c = pl.program_id(1)
    lat = lax.dot_general(
        x_ref[...], we_ref[...], (((1,), (1,)), ((), ())),
        preferred_element_type=jnp.float32,
    ) + be_ref[...][None, :]
    lat_ref[:, pl.ds(c * CHUNK, CHUNK)] = lat

    @pl.when(c == NC - 1)
    def _finish():
        t = jnp.full((BLK, 1), jnp.inf, jnp.float32)
        for _ in range(K):
            m = jnp.full((BLK,), NEG, jnp.float32)
            for j in range(NC):
                part = lat_ref[:, pl.ds(j * CHUNK, CHUNK)]
                m = jnp.maximum(
                    m, jnp.max(jnp.where(part < t, part, NEG), axis=1))
            t = m[:, None]
        for j in range(NC):
            part = lat_ref[:, pl.ds(j * CHUNK, CHUNK)]
            sp_ref[:, pl.ds(j * CHUNK, CHUNK)] = jnp.where(part >= t, part, 0.0)


def _dec_body(sp_ref, wd_ref, bd_ref, out_ref):
    acc = jnp.zeros((BLK, D_IN), jnp.float32)
    for j in range(NC):
        acc += lax.dot_general(
            sp_ref[:, pl.ds(j * CHUNK, CHUNK)],
            wd_ref[:, pl.ds(j * CHUNK, CHUNK)],
            (((1,), (1,)), ((), ())),
            preferred_element_type=jnp.float32,
        )
    out_ref[...] = acc + bd_ref[...][None, :]


def kernel(x, W_enc, b_enc, W_dec, b_dec):
    sparse = pl.pallas_call(
        _enc_body,
        grid=(ROWS // BLK, NC),
        in_specs=[
            pl.BlockSpec((BLK, D_IN), lambda r, c: (r, 0)),
            pl.BlockSpec((CHUNK, D_IN), lambda r, c: (c, 0)),
            pl.BlockSpec((CHUNK,), lambda r, c: (c,)),
        ],
        out_specs=pl.BlockSpec((BLK, D_LAT), lambda r, c: (r, 0)),
        out_shape=jax.ShapeDtypeStruct((ROWS, D_LAT), jnp.float32),
        scratch_shapes=[pltpu.VMEM((BLK, D_LAT), jnp.float32)],
    )(x, W_enc, b_enc)

    return pl.pallas_call(
        _dec_body,
        grid=(ROWS // BLK,),
        in_specs=[
            pl.BlockSpec((BLK, D_LAT), lambda r: (r, 0)),
            pl.BlockSpec((D_IN, D_LAT), lambda r: (0, 0)),
            pl.BlockSpec((D_IN,), lambda r: (0,)),
        ],
        out_specs=pl.BlockSpec((BLK, D_IN), lambda r: (r, 0)),
        out_shape=jax.ShapeDtypeStruct((ROWS, D_IN), jnp.float32),
    )(sparse, W_dec, b_dec)
